# baseline (speedup 1.0000x reference)
import jax
import jax.numpy as jnp
from jax import lax
from jax.experimental import pallas as pl
from jax.experimental.pallas import tpu as pltpu
from jax.experimental.pallas import tpu_sc as plsc

B = 4096
L = 200
D = 768
DW = D // 2
NC = 2
NS = 16
NW = NC * NS
ROWS_PER_W = B // NW
G = 40
CHUNKS_PER_ROW = L // G
NCHUNK = ROWS_PER_W * CHUNKS_PER_ROW
NLANE = 16
NDW = DW // NLANE
INV_L = 1.0 / L


def _sc_body(ids_hbm, tab_hbm, out_hbm, idx_v, buf0, buf1, acc_a, acc_b,
             row_v, sem0, sem1):
    wid = lax.axis_index("s") * NC + lax.axis_index("c")

    pltpu.sync_copy(ids_hbm.at[wid], idx_v)

    def _start(g, buf, sem):
        pltpu.make_async_copy(
            tab_hbm.at[idx_v.at[pl.ds(g * G, G)]], buf, sem).start()

    def _wait(buf, sem):
        pltpu.make_async_copy(
            tab_hbm.at[idx_v.at[pl.ds(0, G)]], buf, sem).wait()

    def _zero_acc():
        def z(d, _):
            sl = pl.ds(d * NLANE, NLANE)
            acc_a[sl] = jnp.zeros((NLANE,), jnp.float32)
            acc_b[sl] = jnp.zeros((NLANE,), jnp.float32)
            return 0
        lax.fori_loop(0, NDW, z, 0)

    def _accum(buf):
        @plsc.parallel_loop(0, NDW, unroll=2)
        def _(d):
            sl = pl.ds(d * NLANE, NLANE)

            def u(k):
                return plsc.unpack(
                    plsc.bitcast(buf[k, sl], jnp.bfloat16),
                    format=plsc.PackFormat.INTERLEAVED,
                    preferred_element_type=jnp.float32)

            a0, b0 = u(0)
            a1, b1 = u(1)
            for k in range(2, G, 2):
                xa, xb = u(k)
                a0 = a0 + xa
                b0 = b0 + xb
                ya, yb = u(k + 1)
                a1 = a1 + ya
                b1 = b1 + yb
            acc_a[sl] = acc_a[sl] + (a0 + a1)
            acc_b[sl] = acc_b[sl] + (b0 + b1)

    def _maybe_finish(cnt, row):
        @pl.when(cnt == CHUNKS_PER_ROW - 1)
        def _():
            def s(d, _):
                sl = pl.ds(d * NLANE, NLANE)
                va = acc_a[sl] * jnp.float32(INV_L)
                vb = acc_b[sl] * jnp.float32(INV_L)
                row_v[sl] = plsc.bitcast(
                    plsc.pack(va, vb, format=plsc.PackFormat.INTERLEAVED),
                    jnp.int32)
                acc_a[sl] = jnp.zeros((NLANE,), jnp.float32)
                acc_b[sl] = jnp.zeros((NLANE,), jnp.float32)
                return 0
            lax.fori_loop(0, NDW, s, 0)
            pltpu.sync_copy(row_v, out_hbm.at[pl.ds(row * DW, DW)])
        done = cnt == CHUNKS_PER_ROW - 1
        return jnp.where(done, 0, cnt + 1), jnp.where(done, row + 1, row)

    _zero_acc()
    _start(0, buf0, sem0)

    def pair(gp, carry):
        cnt, row = carry
        g0 = 2 * gp
        _start(g0 + 1, buf1, sem1)
        _wait(buf0, sem0)
        _accum(buf0)
        cnt, row = _maybe_finish(cnt, row)

        @pl.when(g0 + 2 < NCHUNK)
        def _():
            _start(g0 + 2, buf0, sem0)
        _wait(buf1, sem1)
        _accum(buf1)
        cnt, row = _maybe_finish(cnt, row)
        return cnt, row

    lax.fori_loop(0, NCHUNK // 2, pair,
                  (jnp.int32(0), (wid * ROWS_PER_W).astype(jnp.int32)))


@jax.jit
def kernel(ids, table):
    vocab = table.shape[0]
    tabp = lax.bitcast_convert_type(
        table.astype(jnp.bfloat16).reshape(vocab, DW, 2), jnp.int32)
    ids3 = ids.reshape(NW, NCHUNK * G)
    mesh = plsc.VectorSubcoreMesh(core_axis_name="c", subcore_axis_name="s")
    outp = pl.kernel(
        _sc_body,
        out_type=jax.ShapeDtypeStruct((B * DW,), jnp.int32),
        mesh=mesh,
        compiler_params=pltpu.CompilerParams(needs_layout_passes=False),
        scratch_types=[
            pltpu.VMEM((NCHUNK * G,), jnp.int32),
            pltpu.VMEM((G, DW), jnp.int32),
            pltpu.VMEM((G, DW), jnp.int32),
            pltpu.VMEM((DW,), jnp.float32),
            pltpu.VMEM((DW,), jnp.float32),
            pltpu.VMEM((DW,), jnp.int32),
            pltpu.SemaphoreType.DMA,
            pltpu.SemaphoreType.DMA,
        ],
    )(ids3, tabp)
    outb = lax.bitcast_convert_type(outp.reshape(B, DW), jnp.bfloat16)
    return outb.reshape(B, D).astype(jnp.float32)


# speedup vs baseline: 2.8349x; 2.8349x over previous
import jax
import jax.numpy as jnp
from jax import lax
from jax.experimental import pallas as pl
from jax.experimental.pallas import tpu as pltpu
from jax.experimental.pallas import tpu_sc as plsc

B = 4096
L = 200
D = 768
DW = D // 2
NC = 2
NS = 16
NW = NC * NS
ROWS_PER_W = B // NW
G = 40
CHUNKS_PER_ROW = L // G
NCHUNK = ROWS_PER_W * CHUNKS_PER_ROW
NLANE = 16
NDW = DW // NLANE
INV_L = 1.0 / L
CONV_BLK = 1024


def _conv_body(x_ref, o_ref):
    a = lax.bitcast_convert_type(x_ref[:, :DW], jnp.int32)
    b = lax.bitcast_convert_type(x_ref[:, DW:], jnp.int32)
    ra = a + jnp.int32(0x7FFF) + ((a >> 16) & jnp.int32(1))
    rb = b + jnp.int32(0x7FFF) + ((b >> 16) & jnp.int32(1))
    o_ref[...] = ((ra >> 16) & jnp.int32(0xFFFF)) | (rb & jnp.int32(-65536))


def _sc_body(ids_hbm, tab_hbm, out_hbm, idx_v, buf0, buf1, acc_a, acc_b,
             row_v, sem0, sem1):
    wid = lax.axis_index("s") * NC + lax.axis_index("c")

    pltpu.sync_copy(ids_hbm.at[wid], idx_v)

    def _start(g, buf, sem):
        pltpu.make_async_copy(
            tab_hbm.at[idx_v.at[pl.ds(g * G, G)]], buf, sem).start()

    def _wait(buf, sem):
        pltpu.make_async_copy(
            tab_hbm.at[idx_v.at[pl.ds(0, G)]], buf, sem).wait()

    def _zero_acc():
        def z(d, _):
            sl = pl.ds(d * NLANE, NLANE)
            acc_a[sl] = jnp.zeros((NLANE,), jnp.float32)
            acc_b[sl] = jnp.zeros((NLANE,), jnp.float32)
            return 0
        lax.fori_loop(0, NDW, z, 0)

    def _accum(buf):
        @plsc.parallel_loop(0, NDW, unroll=2)
        def _(d):
            sl = pl.ds(d * NLANE, NLANE)

            def u(k):
                return plsc.unpack(
                    plsc.bitcast(buf[k, sl], jnp.bfloat16),
                    format=plsc.PackFormat.INTERLEAVED,
                    preferred_element_type=jnp.float32)

            a0, b0 = u(0)
            a1, b1 = u(1)
            for k in range(2, G, 2):
                xa, xb = u(k)
                a0 = a0 + xa
                b0 = b0 + xb
                ya, yb = u(k + 1)
                a1 = a1 + ya
                b1 = b1 + yb
            acc_a[sl] = acc_a[sl] + (a0 + a1)
            acc_b[sl] = acc_b[sl] + (b0 + b1)

    def _maybe_finish(cnt, row):
        @pl.when(cnt == CHUNKS_PER_ROW - 1)
        def _():
            def s(d, _):
                sl = pl.ds(d * NLANE, NLANE)
                row_v[sl] = acc_a[sl] * jnp.float32(INV_L)
                row_v[pl.ds(DW + d * NLANE, NLANE)] = (
                    acc_b[sl] * jnp.float32(INV_L))
                acc_a[sl] = jnp.zeros((NLANE,), jnp.float32)
                acc_b[sl] = jnp.zeros((NLANE,), jnp.float32)
                return 0
            lax.fori_loop(0, NDW, s, 0)
            pltpu.sync_copy(row_v, out_hbm.at[pl.ds(row * D, D)])
        done = cnt == CHUNKS_PER_ROW - 1
        return jnp.where(done, 0, cnt + 1), jnp.where(done, row + 1, row)

    _zero_acc()
    _start(0, buf0, sem0)

    def pair(gp, carry):
        cnt, row = carry
        g0 = 2 * gp
        _start(g0 + 1, buf1, sem1)
        _wait(buf0, sem0)
        _accum(buf0)
        cnt, row = _maybe_finish(cnt, row)

        @pl.when(g0 + 2 < NCHUNK)
        def _():
            _start(g0 + 2, buf0, sem0)
        _wait(buf1, sem1)
        _accum(buf1)
        cnt, row = _maybe_finish(cnt, row)
        return cnt, row

    lax.fori_loop(0, NCHUNK // 2, pair,
                  (jnp.int32(0), (wid * ROWS_PER_W).astype(jnp.int32)))


@jax.jit
def kernel(ids, table):
    vocab = table.shape[0]
    nblk = (vocab + CONV_BLK - 1) // CONV_BLK
    tabp = pl.pallas_call(
        _conv_body,
        grid=(nblk,),
        in_specs=[pl.BlockSpec((CONV_BLK, D), lambda i: (i, 0))],
        out_specs=pl.BlockSpec((CONV_BLK, DW), lambda i: (i, 0)),
        out_shape=jax.ShapeDtypeStruct((vocab, DW), jnp.int32),
    )(table)
    ids3 = ids.reshape(NW, NCHUNK * G)
    mesh = plsc.VectorSubcoreMesh(core_axis_name="c", subcore_axis_name="s")
    out = pl.kernel(
        _sc_body,
        out_type=jax.ShapeDtypeStruct((B * D,), jnp.float32),
        mesh=mesh,
        compiler_params=pltpu.CompilerParams(needs_layout_passes=False),
        scratch_types=[
            pltpu.VMEM((NCHUNK * G,), jnp.int32),
            pltpu.VMEM((G, DW), jnp.int32),
            pltpu.VMEM((G, DW), jnp.int32),
            pltpu.VMEM((DW,), jnp.float32),
            pltpu.VMEM((DW,), jnp.float32),
            pltpu.VMEM((D,), jnp.float32),
            pltpu.SemaphoreType.DMA,
            pltpu.SemaphoreType.DMA,
        ],
    )(ids3, tabp)
    return out.reshape(B, D)
